# Initial kernel scaffold; baseline (speedup 1.0000x reference)
import jax
import jax.numpy as jnp
from jax.experimental import pallas as pl


def kernel(atomic_numbers, pair_indices, d_ij, r_ij, emb, emb2_W, emb2_b, proj1_W, proj1_b, proj2_W, proj2_b, proj3_W, proj3_b, lt0_W, lt1_W, lt2_W, ls0_W, ls0_b, ls1_W, ls1_b, ln_g, ln_b):
    raise NotImplementedError("write your pallas kernel here")


# speedup vs baseline: 18.9285x; 18.9285x over previous
import functools
import math

import jax
import jax.numpy as jnp
import numpy as np
from jax.experimental import pallas as pl
from jax.experimental.pallas import tpu as pltpu

H = 64
R = 32
CUTOFF = 5.0
BE = 1000
BN = 1000


def _edge_kernel(za_ref, zb_ref, d_ref, r_ref, WaT_ref, WbT_ref, eb_ref,
                 p1T_ref, p2T_ref, p3T_ref, b1_ref, b2_ref, b3_ref,
                 means_ref, out_ref):
    za = za_ref[...]
    zb = zb_ref[...]
    d = d_ref[...]
    Zij = (jnp.dot(za, WaT_ref[...], preferred_element_type=jnp.float32)
           + jnp.dot(zb, WbT_ref[...], preferred_element_type=jnp.float32)
           + eb_ref[...])
    rcut = 0.5 * (jnp.cos(d * (math.pi / CUTOFF)) + 1.0) * (d < CUTOFF).astype(jnp.float32)
    alpha = 5.0 / CUTOFF
    beta = (2.0 / R * (1.0 - math.exp(-CUTOFF))) ** -2
    expd = jnp.exp(-alpha * d)
    rfv = jnp.exp(-beta * (expd - means_ref[...]) ** 2) * rcut
    p1 = jnp.dot(rfv, p1T_ref[...], preferred_element_type=jnp.float32) + b1_ref[...]
    p2 = jnp.dot(rfv, p2T_ref[...], preferred_element_type=jnp.float32) + b2_ref[...]
    p3 = jnp.dot(rfv, p3T_ref[...], preferred_element_type=jnp.float32) + b3_ref[...]
    C = rcut * Zij
    a1 = C * p1
    a2 = C * p2
    a3 = C * p3
    inv_d = 1.0 / d
    vx = r_ref[:, 0:1] * inv_d
    vy = r_ref[:, 1:2] * inv_d
    vz = r_ref[:, 2:3] * inv_d
    out_ref[:, 0:64] = a1
    out_ref[:, 64:128] = a2 * vx
    out_ref[:, 128:192] = a2 * vy
    out_ref[:, 192:256] = a2 * vz
    out_ref[:, 256:320] = a3 * (vx * vx)
    out_ref[:, 320:384] = a3 * (vx * vy)
    out_ref[:, 384:448] = a3 * (vx * vz)
    out_ref[:, 448:512] = a3 * (vy * vy)
    out_ref[:, 512:576] = a3 * (vy * vz)
    out_ref[:, 576:640] = a3 * (vz * vz)


def _edge_payload(za, zb, d_ij, r_ij, emb2_W, emb2_b, p1T, p2T, p3T,
                  b1, b2, b3, means):
    E = za.shape[0]
    grid = E // BE
    full = lambda shp: pl.BlockSpec(shp, lambda i: (0, 0))
    return pl.pallas_call(
        _edge_kernel,
        grid=(grid,),
        in_specs=[
            pl.BlockSpec((BE, H), lambda i: (i, 0)),
            pl.BlockSpec((BE, H), lambda i: (i, 0)),
            pl.BlockSpec((BE, 1), lambda i: (i, 0)),
            pl.BlockSpec((BE, 3), lambda i: (i, 0)),
            full((H, H)), full((H, H)), full((1, H)),
            full((R, H)), full((R, H)), full((R, H)),
            full((1, H)), full((1, H)), full((1, H)),
            full((1, R)),
        ],
        out_specs=pl.BlockSpec((BE, 10 * H), lambda i: (i, 0)),
        out_shape=jax.ShapeDtypeStruct((E, 10 * H), jnp.float32),
    )(za, zb, d_ij, r_ij, emb2_W[:, :H].T, emb2_W[:, H:].T, emb2_b[None],
      p1T, p2T, p3T, b1[None], b2[None], b3[None], means[None])


def _silu(x):
    return x * (1.0 / (1.0 + jnp.exp(-x)))


def _atom_kernel(s_ref, lt0T_ref, lt1T_ref, lt2T_ref, ls0T_ref, ls0b_ref,
                 ls1T_ref, ls1b_ref, lng_ref, lnb_ref, out_ref):
    s1 = s_ref[:, 0:64]
    w0 = s_ref[:, 64:128]
    w1 = s_ref[:, 128:192]
    w2 = s_ref[:, 192:256]
    Q0 = s_ref[:, 256:320]
    Q1 = s_ref[:, 320:384]
    Q2 = s_ref[:, 384:448]
    Q3 = s_ref[:, 448:512]
    Q4 = s_ref[:, 512:576]
    Q5 = s_ref[:, 576:640]
    trQ3 = (Q0 + Q3 + Q5) * (100.0 / 3.0)
    T00 = s1 + 100.0 * Q0 - trQ3
    T11 = s1 + 100.0 * Q3 - trQ3
    T22 = s1 + 100.0 * Q5 - trQ3
    off01 = 100.0 * Q1
    off02 = 100.0 * Q2
    off12 = 100.0 * Q4
    sw0 = 10.0 * w0
    sw1 = 10.0 * w1
    sw2 = 10.0 * w2
    tnorm = (T00 * T00 + T11 * T11 + T22 * T22
             + (off01 - sw2) ** 2 + (off01 + sw2) ** 2
             + (off02 + sw1) ** 2 + (off02 - sw1) ** 2
             + (off12 - sw0) ** 2 + (off12 + sw0) ** 2)
    mu = jnp.mean(tnorm, axis=-1, keepdims=True)
    var = jnp.mean((tnorm - mu) ** 2, axis=-1, keepdims=True)
    nrm = (tnorm - mu) * jax.lax.rsqrt(var + 1e-5) * lng_ref[...] + lnb_ref[...]
    y0 = _silu(jnp.dot(nrm, ls0T_ref[...], preferred_element_type=jnp.float32)
               + ls0b_ref[...])
    y1 = _silu(jnp.dot(y0, ls1T_ref[...], preferred_element_type=jnp.float32)
               + ls1b_ref[...])
    n0 = y1[:, 0:64]
    n1 = y1[:, 64:128]
    n2 = y1[:, 128:192]
    lt0T = lt0T_ref[...]
    lt1T = lt1T_ref[...]
    lt2T = lt2T_ref[...]
    s1p = jnp.dot(s1, lt0T, preferred_element_type=jnp.float32)
    w0p = jnp.dot(w0, lt1T, preferred_element_type=jnp.float32)
    w1p = jnp.dot(w1, lt1T, preferred_element_type=jnp.float32)
    w2p = jnp.dot(w2, lt1T, preferred_element_type=jnp.float32)
    Q0p = jnp.dot(Q0, lt2T, preferred_element_type=jnp.float32)
    Q1p = jnp.dot(Q1, lt2T, preferred_element_type=jnp.float32)
    Q2p = jnp.dot(Q2, lt2T, preferred_element_type=jnp.float32)
    Q3p = jnp.dot(Q3, lt2T, preferred_element_type=jnp.float32)
    Q4p = jnp.dot(Q4, lt2T, preferred_element_type=jnp.float32)
    Q5p = jnp.dot(Q5, lt2T, preferred_element_type=jnp.float32)
    trQp3 = (Q0p + Q3p + Q5p) * (100.0 / 3.0)
    diag = n0 * s1p
    o01 = n2 * 100.0 * Q1p
    o02 = n2 * 100.0 * Q2p
    o12 = n2 * 100.0 * Q4p
    sw0p = n1 * 10.0 * w0p
    sw1p = n1 * 10.0 * w1p
    sw2p = n1 * 10.0 * w2p
    out_ref[:, 0, :] = diag + n2 * (100.0 * Q0p - trQp3)
    out_ref[:, 1, :] = o01 - sw2p
    out_ref[:, 2, :] = o02 + sw1p
    out_ref[:, 3, :] = o01 + sw2p
    out_ref[:, 4, :] = diag + n2 * (100.0 * Q3p - trQp3)
    out_ref[:, 5, :] = o12 - sw0p
    out_ref[:, 6, :] = o02 - sw1p
    out_ref[:, 7, :] = o12 + sw0p
    out_ref[:, 8, :] = diag + n2 * (100.0 * Q5p - trQp3)


def _atom_stage(sums, lt0_W, lt1_W, lt2_W, ls0_W, ls0_b, ls1_W, ls1_b,
                ln_g, ln_b):
    N = sums.shape[0]
    grid = N // BN
    perm = np.concatenate([np.arange(0, 3 * H, 3), np.arange(1, 3 * H, 3),
                           np.arange(2, 3 * H, 3)])
    ls1pT = ls1_W[perm].T
    ls1pb = ls1_b[perm]
    full = lambda shp: pl.BlockSpec(shp, lambda i: tuple(0 for _ in shp))
    out9 = pl.pallas_call(
        _atom_kernel,
        grid=(grid,),
        in_specs=[
            pl.BlockSpec((BN, 10 * H), lambda i: (i, 0)),
            full((H, H)), full((H, H)), full((H, H)),
            full((H, 2 * H)), full((1, 2 * H)),
            full((2 * H, 3 * H)), full((1, 3 * H)),
            full((1, H)), full((1, H)),
        ],
        out_specs=pl.BlockSpec((BN, 9, H), lambda i: (i, 0, 0)),
        out_shape=jax.ShapeDtypeStruct((N, 9, H), jnp.float32),
    )(sums, lt0_W.T, lt1_W.T, lt2_W.T, ls0_W.T, ls0_b[None], ls1pT,
      ls1pb[None], ln_g[None], ln_b[None])
    return jnp.transpose(out9, (0, 2, 1)).reshape(N, H, 3, 3)


def kernel(atomic_numbers, pair_indices, d_ij, r_ij, emb, emb2_W, emb2_b,
           proj1_W, proj1_b, proj2_W, proj2_b, proj3_W, proj3_b,
           lt0_W, lt1_W, lt2_W, ls0_W, ls0_b, ls1_W, ls1_b, ln_g, ln_b):
    N = atomic_numbers.shape[0]
    p0, p1 = pair_indices[0], pair_indices[1]
    zi = jnp.take(emb, atomic_numbers, axis=0)
    za = jnp.take(zi, p0, axis=0)
    zb = jnp.take(zi, p1, axis=0)
    means = jnp.linspace(np.exp(-CUTOFF), 1.0, R).astype(jnp.float32)
    payload = _edge_payload(za, zb, d_ij, r_ij, emb2_W, emb2_b,
                            proj1_W.T, proj2_W.T, proj3_W.T,
                            proj1_b, proj2_b, proj3_b, means)
    sums = jax.ops.segment_sum(payload, p0, num_segments=N)
    return _atom_stage(sums, lt0_W, lt1_W, lt2_W, ls0_W, ls0_b,
                       ls1_W, ls1_b, ln_g, ln_b)
